# baseline (speedup 1.0000x reference)
import jax
import jax.numpy as jnp
from jax import lax
from jax.experimental import pallas as pl
from jax.experimental.pallas import tpu as pltpu
from jax.experimental.pallas import tpu_sc as plsc

D_MODEL = 2048
SCALE = 12.0
N_TOKENS = 4 * 4096

NUM_CORES = 2
NUM_SUBCORES = 16
LANES = 16
NW = NUM_CORES * NUM_SUBCORES
B_PER_W = N_TOKENS // NW
CHUNK = 16
NCH = B_PER_W // CHUNK
VECS_PER_ROW = D_MODEL // LANES


def _emb_body(ids_hbm, table_hbm, out_hbm, idx_v, buf_v, gsem, ssem):
    wid = lax.axis_index("s") * NUM_CORES + lax.axis_index("c")
    base = wid * B_PER_W
    pltpu.sync_copy(ids_hbm.at[pl.ds(base, B_PER_W)], idx_v)

    @pl.loop(0, NCH)
    def _chunk(c):
        row0 = c * CHUNK
        pltpu.async_copy(
            table_hbm.at[idx_v.at[pl.ds(row0, CHUNK)]], buf_v, gsem
        ).wait()

        @pl.loop(0, CHUNK)
        def _row(r):
            @pl.loop(0, VECS_PER_ROW)
            def _vec(i):
                sl = pl.ds(i * LANES, LANES)
                buf_v[r, sl] = buf_v[r, sl] * SCALE

        pltpu.async_copy(
            buf_v, out_hbm.at[pl.ds(base + row0, CHUNK)], ssem
        ).wait()


@jax.jit
def _embed(ids_flat, embed_table):
    mesh = plsc.VectorSubcoreMesh(
        core_axis_name="c", subcore_axis_name="s",
        num_cores=NUM_CORES, num_subcores=NUM_SUBCORES,
    )
    run = pl.kernel(
        _emb_body,
        out_type=jax.ShapeDtypeStruct((N_TOKENS, D_MODEL), jnp.float32),
        mesh=mesh,
        scratch_types=[
            pltpu.VMEM((B_PER_W,), jnp.int32),
            pltpu.VMEM((CHUNK, D_MODEL), jnp.float32),
            pltpu.SemaphoreType.DMA,
            pltpu.SemaphoreType.DMA,
        ],
    )
    return run(ids_flat, embed_table)


def kernel(input_ids, embed_table):
    b, s = input_ids.shape
    ids_flat = input_ids.reshape(-1).astype(jnp.int32)
    out = _embed(ids_flat, embed_table)
    return out.reshape(b, s, D_MODEL)


# speedup vs baseline: 1.3407x; 1.3407x over previous
import jax
import jax.numpy as jnp
from jax import lax
from jax.experimental import pallas as pl
from jax.experimental.pallas import tpu as pltpu
from jax.experimental.pallas import tpu_sc as plsc

D_MODEL = 2048
SCALE = 12.0
N_TOKENS = 4 * 4096

NUM_CORES = 2
NUM_SUBCORES = 16
LANES = 16
NW = NUM_CORES * NUM_SUBCORES
B_PER_W = N_TOKENS // NW
CHUNK = 16
NCH = B_PER_W // CHUNK
VECS_PER_ROW = D_MODEL // LANES


def _emb_body(ids_hbm, table_hbm, out_hbm, idx_v, buf_v, gsem, ssem):
    wid = lax.axis_index("s") * NUM_CORES + lax.axis_index("c")
    base = wid * B_PER_W
    pltpu.sync_copy(ids_hbm.at[pl.ds(base, B_PER_W)], idx_v)

    @pl.loop(0, NCH)
    def _chunk(c):
        row0 = c * CHUNK
        pltpu.async_copy(
            table_hbm.at[idx_v.at[pl.ds(row0, CHUNK)]], buf_v, gsem
        ).wait()

        @pl.loop(0, VECS_PER_ROW, unroll=2)
        def _vec(i):
            sl = pl.ds(i * LANES, LANES)
            for r in range(CHUNK):
                buf_v[r, sl] = buf_v[r, sl] * SCALE

        pltpu.async_copy(
            buf_v, out_hbm.at[pl.ds(base + row0, CHUNK)], ssem
        ).wait()


@jax.jit
def _embed(ids_flat, embed_table):
    mesh = plsc.VectorSubcoreMesh(
        core_axis_name="c", subcore_axis_name="s",
        num_cores=NUM_CORES, num_subcores=NUM_SUBCORES,
    )
    run = pl.kernel(
        _emb_body,
        out_type=jax.ShapeDtypeStruct((N_TOKENS, D_MODEL), jnp.float32),
        mesh=mesh,
        scratch_types=[
            pltpu.VMEM((B_PER_W,), jnp.int32),
            pltpu.VMEM((CHUNK, D_MODEL), jnp.float32),
            pltpu.SemaphoreType.DMA,
            pltpu.SemaphoreType.DMA,
        ],
    )
    return run(ids_flat, embed_table)


def kernel(input_ids, embed_table):
    b, s = input_ids.shape
    ids_flat = input_ids.reshape(-1).astype(jnp.int32)
    out = _embed(ids_flat, embed_table)
    return out.reshape(b, s, D_MODEL)


# speedup vs baseline: 1.4322x; 1.0682x over previous
import jax
import jax.numpy as jnp
from jax import lax
from jax.experimental import pallas as pl
from jax.experimental.pallas import tpu as pltpu
from jax.experimental.pallas import tpu_sc as plsc

D_MODEL = 2048
SCALE = 12.0
N_TOKENS = 4 * 4096

NUM_CORES = 2
NUM_SUBCORES = 16
LANES = 16
NW = NUM_CORES * NUM_SUBCORES
B_PER_W = N_TOKENS // NW
CHUNK = 16
NCH = B_PER_W // CHUNK
VECS_PER_ROW = D_MODEL // LANES


def _emb_body(ids_hbm, table_hbm, out_hbm, idx_v,
              buf0, buf1, gsem, s0, s1):
    bufs = (buf0, buf1)
    ssems = (s0, s1)

    wid = lax.axis_index("s") * NUM_CORES + lax.axis_index("c")
    base = wid * B_PER_W
    pltpu.sync_copy(ids_hbm.at[pl.ds(base, B_PER_W)], idx_v)

    def gather(c, b):
        pltpu.async_copy(
            table_hbm.at[idx_v.at[pl.ds(c * CHUNK, CHUNK)]], bufs[b], gsem
        ).wait()

    def scale(b):
        buf = bufs[b]

        @pl.loop(0, VECS_PER_ROW, unroll=2)
        def _vec(i):
            sl = pl.ds(i * LANES, LANES)
            for r in range(CHUNK):
                buf[r, sl] = buf[r, sl] * SCALE

    def start_scatter(c, b):
        pltpu.async_copy(
            bufs[b], out_hbm.at[pl.ds(base + c * CHUNK, CHUNK)], ssems[b])

    def drain_scatter(b):
        pltpu.make_async_copy(
            bufs[b], out_hbm.at[pl.ds(base, CHUNK)], ssems[b]).wait()

    for b in range(2):
        gather(b, b)
        scale(b)
        start_scatter(b, b)

    @pl.loop(1, NCH // 2)
    def _round(g):
        for b in range(2):
            c = g * 2 + b
            drain_scatter(b)
            gather(c, b)
            scale(b)
            start_scatter(c, b)

    for b in range(2):
        drain_scatter(b)


@jax.jit
def _embed(ids_flat, embed_table):
    mesh = plsc.VectorSubcoreMesh(
        core_axis_name="c", subcore_axis_name="s",
        num_cores=NUM_CORES, num_subcores=NUM_SUBCORES,
    )
    run = pl.kernel(
        _emb_body,
        out_type=jax.ShapeDtypeStruct((N_TOKENS, D_MODEL), jnp.float32),
        mesh=mesh,
        scratch_types=[
            pltpu.VMEM((B_PER_W,), jnp.int32),
            pltpu.VMEM((CHUNK, D_MODEL), jnp.float32),
            pltpu.VMEM((CHUNK, D_MODEL), jnp.float32),
            pltpu.SemaphoreType.DMA,
            pltpu.SemaphoreType.DMA,
            pltpu.SemaphoreType.DMA,
        ],
    )
    return run(ids_flat, embed_table)


def kernel(input_ids, embed_table):
    b, s = input_ids.shape
    ids_flat = input_ids.reshape(-1).astype(jnp.int32)
    out = _embed(ids_flat, embed_table)
    return out.reshape(b, s, D_MODEL)


# speedup vs baseline: 1.6797x; 1.1728x over previous
import jax
import jax.numpy as jnp
from jax import lax
from jax.experimental import pallas as pl
from jax.experimental.pallas import tpu as pltpu
from jax.experimental.pallas import tpu_sc as plsc

D_MODEL = 2048
SCALE = 12.0
N_TOKENS = 4 * 4096

NUM_CORES = 2
NUM_SUBCORES = 16
LANES = 16
NW = NUM_CORES * NUM_SUBCORES
B_PER_W = N_TOKENS // NW
CHUNK = 16
NCH = B_PER_W // CHUNK
VECS_PER_ROW = D_MODEL // LANES


def _emb_body(ids_hbm, table_hbm, out_hbm, idx_v,
              buf0, buf1, g0, g1, s0, s1):
    bufs = (buf0, buf1)
    gsems = (g0, g1)
    ssems = (s0, s1)

    wid = lax.axis_index("s") * NUM_CORES + lax.axis_index("c")
    base = wid * B_PER_W
    pltpu.sync_copy(ids_hbm.at[pl.ds(base, B_PER_W)], idx_v)

    def start_gather(c, b):
        pltpu.async_copy(
            table_hbm.at[idx_v.at[pl.ds(c * CHUNK, CHUNK)]], bufs[b], gsems[b])

    def drain_gather(b):
        pltpu.make_async_copy(
            table_hbm.at[idx_v.at[pl.ds(0, CHUNK)]], bufs[b], gsems[b]).wait()

    def scale(b):
        buf = bufs[b]

        @pl.loop(0, VECS_PER_ROW, unroll=2)
        def _vec(i):
            sl = pl.ds(i * LANES, LANES)
            for r in range(CHUNK):
                buf[r, sl] = buf[r, sl] * SCALE

    def start_scatter(c, b):
        pltpu.async_copy(
            bufs[b], out_hbm.at[pl.ds(base + c * CHUNK, CHUNK)], ssems[b])

    def drain_scatter(b):
        pltpu.make_async_copy(
            bufs[b], out_hbm.at[pl.ds(base, CHUNK)], ssems[b]).wait()

    start_gather(0, 0)

    drain_gather(0)
    start_gather(1, 1)
    scale(0)
    start_scatter(0, 0)

    @pl.loop(1, NCH // 2)
    def _pair(g):
        for b, c in ((1, 2 * g - 1), (0, 2 * g)):
            drain_gather(b)
            drain_scatter(1 - b)
            start_gather(c + 1, 1 - b)
            scale(b)
            start_scatter(c, b)

    drain_gather(1)
    scale(1)
    start_scatter(NCH - 1, 1)
    drain_scatter(0)
    drain_scatter(1)


@jax.jit
def _embed(ids_flat, embed_table):
    mesh = plsc.VectorSubcoreMesh(
        core_axis_name="c", subcore_axis_name="s",
        num_cores=NUM_CORES, num_subcores=NUM_SUBCORES,
    )
    run = pl.kernel(
        _emb_body,
        out_type=jax.ShapeDtypeStruct((N_TOKENS, D_MODEL), jnp.float32),
        mesh=mesh,
        scratch_types=[
            pltpu.VMEM((B_PER_W,), jnp.int32),
            pltpu.VMEM((CHUNK, D_MODEL), jnp.float32),
            pltpu.VMEM((CHUNK, D_MODEL), jnp.float32),
            pltpu.SemaphoreType.DMA,
            pltpu.SemaphoreType.DMA,
            pltpu.SemaphoreType.DMA,
            pltpu.SemaphoreType.DMA,
        ],
    )
    return run(ids_flat, embed_table)


def kernel(input_ids, embed_table):
    b, s = input_ids.shape
    ids_flat = input_ids.reshape(-1).astype(jnp.int32)
    out = _embed(ids_flat, embed_table)
    return out.reshape(b, s, D_MODEL)


# speedup vs baseline: 2.8160x; 1.6765x over previous
import jax
import jax.numpy as jnp
from jax import lax
from jax.experimental import pallas as pl
from jax.experimental.pallas import tpu as pltpu
from jax.experimental.pallas import tpu_sc as plsc

D_MODEL = 2048
SCALE = 12.0
N_TOKENS = 4 * 4096

NUM_CORES = 2
NUM_SUBCORES = 16
LANES = 16
NW = NUM_CORES * NUM_SUBCORES
B_PER_W = N_TOKENS // NW
CHUNK = 8
NBUF = 4
NCH = B_PER_W // CHUNK
VECS_PER_ROW = D_MODEL // LANES


def _emb_body(ids_hbm, table_hbm, out_hbm, idx_v,
              b0, b1, b2, b3, g0, g1, g2, g3, s0, s1, s2, s3):
    bufs = (b0, b1, b2, b3)
    gsems = (g0, g1, g2, g3)
    ssems = (s0, s1, s2, s3)

    wid = lax.axis_index("s") * NUM_CORES + lax.axis_index("c")
    base = wid * B_PER_W
    pltpu.sync_copy(ids_hbm.at[pl.ds(base, B_PER_W)], idx_v)

    def start_gather(c, b):
        pltpu.async_copy(
            table_hbm.at[idx_v.at[pl.ds(c * CHUNK, CHUNK)]], bufs[b], gsems[b])

    def drain_gather(b):
        pltpu.make_async_copy(
            table_hbm.at[idx_v.at[pl.ds(0, CHUNK)]], bufs[b], gsems[b]).wait()

    def scale(b):
        buf = bufs[b]

        @pl.loop(0, VECS_PER_ROW, unroll=2)
        def _vec(i):
            sl = pl.ds(i * LANES, LANES)
            for r in range(CHUNK):
                buf[r, sl] = buf[r, sl] * SCALE

    def start_scatter(c, b):
        pltpu.async_copy(
            bufs[b], out_hbm.at[pl.ds(base + c * CHUNK, CHUNK)], ssems[b])

    def drain_scatter(b):
        pltpu.make_async_copy(
            bufs[b], out_hbm.at[pl.ds(base, CHUNK)], ssems[b]).wait()

    def step(c, b, drain_s, next_g):
        drain_gather(b)
        if drain_s:
            drain_scatter((b + 2) % NBUF)
        if next_g:
            start_gather(c + 2, (b + 2) % NBUF)
        scale(b)
        start_scatter(c, b)

    start_gather(0, 0)
    start_gather(1, 1)
    step(0, 0, False, True)
    step(1, 1, False, True)
    step(2, 2, True, True)
    step(3, 3, True, True)

    @pl.loop(1, NCH // NBUF - 1)
    def _round(g):
        for b in range(NBUF):
            step(g * NBUF + b, b, True, True)

    last = NCH - NBUF
    step(last + 0, 0, True, True)
    step(last + 1, 1, True, True)
    step(last + 2, 2, True, False)
    step(last + 3, 3, True, False)
    drain_scatter(2)
    drain_scatter(3)


@jax.jit
def _embed(ids_flat, embed_table):
    mesh = plsc.VectorSubcoreMesh(
        core_axis_name="c", subcore_axis_name="s",
        num_cores=NUM_CORES, num_subcores=NUM_SUBCORES,
    )
    run = pl.kernel(
        _emb_body,
        out_type=jax.ShapeDtypeStruct((N_TOKENS, D_MODEL), jnp.float32),
        mesh=mesh,
        scratch_types=(
            [pltpu.VMEM((B_PER_W,), jnp.int32)]
            + [pltpu.VMEM((CHUNK, D_MODEL), jnp.float32)] * NBUF
            + [pltpu.SemaphoreType.DMA] * (2 * NBUF)
        ),
    )
    return run(ids_flat, embed_table)


def kernel(input_ids, embed_table):
    b, s = input_ids.shape
    ids_flat = input_ids.reshape(-1).astype(jnp.int32)
    out = _embed(ids_flat, embed_table)
    return out.reshape(b, s, D_MODEL)
